# Initial kernel scaffold; baseline (speedup 1.0000x reference)
import jax
import jax.numpy as jnp
from jax.experimental import pallas as pl


def kernel(seg_features, Wq, bq, Wk, bk, Wv, bv, Wc, bc, ln_a, ln_b):
    raise NotImplementedError("write your pallas kernel here")


# speedup vs baseline: 1.2016x; 1.2016x over previous
import math

import jax
import jax.numpy as jnp
from jax.experimental import pallas as pl
from jax.experimental.pallas import tpu as pltpu

_B, _S, _DM, _DP, _H, _C = 64, 300, 1024, 128, 4, 3862


def _pool_kernel(seg_ref, w_ref, b_ref,
                 scores_ref, attnw_ref, ws0_ref, ws1_ref, ws2_ref, ws3_ref):
    seg = seg_ref[0]
    proj = jnp.dot(seg, w_ref[...],
                   preferred_element_type=jnp.float32) + b_ref[...]
    k = proj[:, 4 * _DP:5 * _DP]
    v = proj[:, 5 * _DP:6 * _DP]
    scale = 1.0 / math.sqrt(_DP)

    kb = k.astype(jnp.bfloat16).astype(jnp.float32)
    score = None
    for h in range(_H):
        qh = proj[:, h * _DP:(h + 1) * _DP]
        qb = qh.astype(jnp.bfloat16).astype(jnp.float32)
        sh = jnp.sum(qb * kb, axis=1, keepdims=True) * scale
        onehot = (jax.lax.broadcasted_iota(jnp.int32, (1, _H), 1) == h
                  ).astype(jnp.float32)
        contrib = sh * onehot
        score = contrib if score is None else score + contrib
    scores_ref[0] = score

    m = jnp.max(score, axis=0, keepdims=True)
    e = jnp.exp(score - m)
    p = e / jnp.sum(e, axis=0, keepdims=True)
    attnw_ref[0] = p

    ws = jax.lax.dot_general(p, v, (((0,), (0,)), ((), ())),
                             preferred_element_type=jnp.float32)
    ws = jnp.maximum(ws, 0.0)
    ws0_ref[...] = ws[0:1, :].reshape(1, 1, _DP)
    ws1_ref[...] = ws[1:2, :].reshape(1, 1, _DP)
    ws2_ref[...] = ws[2:3, :].reshape(1, 1, _DP)
    ws3_ref[...] = ws[3:4, :].reshape(1, 1, _DP)


def _head_kernel(ws0_ref, ws1_ref, ws2_ref, ws3_ref, wct_ref, bc_ref,
                 lna_ref, lnb_ref, probs_ref, idc_ref, closs_ref):
    wct = wct_ref[...]
    bc = bc_ref[...]
    lns = []
    for ws_ref in (ws0_ref, ws1_ref, ws2_ref, ws3_ref):
        wsr = ws_ref[...].reshape(-1, _DP)
        logits = jnp.dot(wsr, wct,
                         preferred_element_type=jnp.float32) + bc
        mean = jnp.mean(logits, axis=1, keepdims=True)
        xc = logits - mean
        var = jnp.sum(xc * xc, axis=1, keepdims=True) / (_C - 1)
        std = jnp.sqrt(var)
        lns.append(lna_ref[...] * xc / (std + 1e-6) + lnb_ref[...])
    vmax = jnp.maximum(jnp.maximum(lns[0], lns[1]),
                       jnp.maximum(lns[2], lns[3]))
    probs_ref[...] = jax.nn.sigmoid(vmax)
    idc = jnp.where(lns[0] == vmax, 0,
                    jnp.where(lns[1] == vmax, 1,
                              jnp.where(lns[2] == vmax, 2, 3)))
    idc_ref[...] = idc.astype(jnp.int32)

    wsum = jnp.sum(wct, axis=0, keepdims=True) + bc
    cm = jnp.max(wsum, axis=1, keepdims=True)
    ce = jnp.exp(wsum - cm)
    cp = ce / jnp.sum(ce, axis=1, keepdims=True)
    cmean = jnp.mean(cp, axis=1, keepdims=True)
    cd = cp - cmean
    cstd = jnp.sqrt(jnp.sum(cd * cd, axis=1, keepdims=True) / (_C - 1))
    closs_ref[...] = (float(_B) * jnp.clip(cstd, 1e-9, 1e9)).reshape(1, 1, 1)


def kernel(seg_features, Wq, bq, Wk, bk, Wv, bv, Wc, bc, ln_a, ln_b):
    w_all = jnp.concatenate([Wq.reshape(_H * _DP, _DM), Wk, Wv], axis=0).T
    b_all = jnp.concatenate([bq.reshape(_H * _DP), bk, bv]).reshape(1, 6 * _DP)
    wct = Wc.T
    bc2 = bc.reshape(1, _C)
    lna2 = ln_a.reshape(1, _C)
    lnb2 = ln_b.reshape(1, _C)

    ws_sds = jax.ShapeDtypeStruct((_B, 1, _DP), jnp.float32)
    scores_b, attnw_b, ws0, ws1, ws2, ws3 = pl.pallas_call(
        _pool_kernel,
        grid=(_B,),
        in_specs=[
            pl.BlockSpec((1, _S, _DM), lambda b: (b, 0, 0)),
            pl.BlockSpec((_DM, 6 * _DP), lambda b: (0, 0)),
            pl.BlockSpec((1, 6 * _DP), lambda b: (0, 0)),
        ],
        out_specs=[
            pl.BlockSpec((1, _S, _H), lambda b: (b, 0, 0)),
            pl.BlockSpec((1, _S, _H), lambda b: (b, 0, 0)),
            pl.BlockSpec((1, 1, _DP), lambda b: (b, 0, 0)),
            pl.BlockSpec((1, 1, _DP), lambda b: (b, 0, 0)),
            pl.BlockSpec((1, 1, _DP), lambda b: (b, 0, 0)),
            pl.BlockSpec((1, 1, _DP), lambda b: (b, 0, 0)),
        ],
        out_shape=[
            jax.ShapeDtypeStruct((_B, _S, _H), jnp.float32),
            jax.ShapeDtypeStruct((_B, _S, _H), jnp.float32),
            ws_sds, ws_sds, ws_sds, ws_sds,
        ],
        compiler_params=pltpu.CompilerParams(
            dimension_semantics=("parallel",)),
        name="attn_pool",
    )(seg_features, w_all, b_all)

    nb = _B // 2
    probs, idc, closs = pl.pallas_call(
        _head_kernel,
        grid=(2,),
        in_specs=[
            pl.BlockSpec((nb, 1, _DP), lambda i: (i, 0, 0)),
            pl.BlockSpec((nb, 1, _DP), lambda i: (i, 0, 0)),
            pl.BlockSpec((nb, 1, _DP), lambda i: (i, 0, 0)),
            pl.BlockSpec((nb, 1, _DP), lambda i: (i, 0, 0)),
            pl.BlockSpec((_DP, _C), lambda i: (0, 0)),
            pl.BlockSpec((1, _C), lambda i: (0, 0)),
            pl.BlockSpec((1, _C), lambda i: (0, 0)),
            pl.BlockSpec((1, _C), lambda i: (0, 0)),
        ],
        out_specs=[
            pl.BlockSpec((nb, _C), lambda i: (i, 0)),
            pl.BlockSpec((nb, _C), lambda i: (i, 0)),
            pl.BlockSpec((1, 1, 1), lambda i: (i, 0, 0)),
        ],
        out_shape=[
            jax.ShapeDtypeStruct((_B, _C), jnp.float32),
            jax.ShapeDtypeStruct((_B, _C), jnp.int32),
            jax.ShapeDtypeStruct((2, 1, 1), jnp.float32),
        ],
        compiler_params=pltpu.CompilerParams(
            dimension_semantics=("parallel",)),
        name="classifier_head",
    )(ws0, ws1, ws2, ws3, wct, bc2, lna2, lnb2)

    vid_probs = probs
    attn_idc = idc
    conv_loss = closs[0, 0, 0]
    return (vid_probs, attn_idc, scores_b, attnw_b, conv_loss)


# speedup vs baseline: 1.2282x; 1.0221x over previous
import math

import jax
import jax.numpy as jnp
from jax.experimental import pallas as pl
from jax.experimental.pallas import tpu as pltpu

_B, _S, _DM, _DP, _H, _C = 64, 300, 1024, 128, 4, 3862
_NB = 4


def _pool_kernel(seg_ref, w_ref, b_ref,
                 scores_ref, attnw_ref, ws0_ref, ws1_ref, ws2_ref, ws3_ref):
    scale = 1.0 / math.sqrt(_DP)
    ws_refs = (ws0_ref, ws1_ref, ws2_ref, ws3_ref)
    for i in range(_NB):
        seg = seg_ref[i]
        proj = jnp.dot(seg, w_ref[...],
                       preferred_element_type=jnp.float32) + b_ref[...]
        k = proj[:, 4 * _DP:5 * _DP]
        v = proj[:, 5 * _DP:6 * _DP]

        kb = k.astype(jnp.bfloat16).astype(jnp.float32)
        score = None
        for h in range(_H):
            qh = proj[:, h * _DP:(h + 1) * _DP]
            qb = qh.astype(jnp.bfloat16).astype(jnp.float32)
            sh = jnp.sum(qb * kb, axis=1, keepdims=True) * scale
            onehot = (jax.lax.broadcasted_iota(jnp.int32, (1, _H), 1) == h
                      ).astype(jnp.float32)
            contrib = sh * onehot
            score = contrib if score is None else score + contrib
        scores_ref[i] = score

        m = jnp.max(score, axis=0, keepdims=True)
        e = jnp.exp(score - m)
        p = e / jnp.sum(e, axis=0, keepdims=True)
        attnw_ref[i] = p

        ws = jax.lax.dot_general(p, v, (((0,), (0,)), ((), ())),
                                 preferred_element_type=jnp.float32)
        ws = jnp.maximum(ws, 0.0)
        for h in range(_H):
            ws_refs[h][i] = ws[h:h + 1, :]


def _head_kernel(ws0_ref, ws1_ref, ws2_ref, ws3_ref, wct_ref, bc_ref,
                 lna_ref, lnb_ref, probs_ref, idc_ref, closs_ref):
    wct = wct_ref[...]
    bc = bc_ref[...]
    lns = []
    for ws_ref in (ws0_ref, ws1_ref, ws2_ref, ws3_ref):
        wsr = ws_ref[...].reshape(-1, _DP)
        logits = jnp.dot(wsr, wct,
                         preferred_element_type=jnp.float32) + bc
        mean = jnp.mean(logits, axis=1, keepdims=True)
        xc = logits - mean
        var = jnp.sum(xc * xc, axis=1, keepdims=True) / (_C - 1)
        std = jnp.sqrt(var)
        lns.append(lna_ref[...] * xc / (std + 1e-6) + lnb_ref[...])
    vmax = jnp.maximum(jnp.maximum(lns[0], lns[1]),
                       jnp.maximum(lns[2], lns[3]))
    probs_ref[...] = jax.nn.sigmoid(vmax)
    idc = jnp.where(lns[0] == vmax, 0,
                    jnp.where(lns[1] == vmax, 1,
                              jnp.where(lns[2] == vmax, 2, 3)))
    idc_ref[...] = idc.astype(jnp.int32)

    wsum = jnp.sum(wct, axis=0, keepdims=True) + bc
    cm = jnp.max(wsum, axis=1, keepdims=True)
    ce = jnp.exp(wsum - cm)
    cp = ce / jnp.sum(ce, axis=1, keepdims=True)
    cmean = jnp.mean(cp, axis=1, keepdims=True)
    cd = cp - cmean
    cstd = jnp.sqrt(jnp.sum(cd * cd, axis=1, keepdims=True) / (_C - 1))
    closs_ref[...] = (float(_B) * jnp.clip(cstd, 1e-9, 1e9)).reshape(1, 1, 1)


def kernel(seg_features, Wq, bq, Wk, bk, Wv, bv, Wc, bc, ln_a, ln_b):
    w_all = jnp.concatenate([Wq.reshape(_H * _DP, _DM), Wk, Wv], axis=0).T
    b_all = jnp.concatenate([bq.reshape(_H * _DP), bk, bv]).reshape(1, 6 * _DP)
    wct = Wc.T
    bc2 = bc.reshape(1, _C)
    lna2 = ln_a.reshape(1, _C)
    lnb2 = ln_b.reshape(1, _C)

    ws_sds = jax.ShapeDtypeStruct((_B, 1, _DP), jnp.float32)
    scores_b, attnw_b, ws0, ws1, ws2, ws3 = pl.pallas_call(
        _pool_kernel,
        grid=(_B // _NB,),
        in_specs=[
            pl.BlockSpec((_NB, _S, _DM), lambda b: (b, 0, 0)),
            pl.BlockSpec((_DM, 6 * _DP), lambda b: (0, 0)),
            pl.BlockSpec((1, 6 * _DP), lambda b: (0, 0)),
        ],
        out_specs=[
            pl.BlockSpec((_NB, _S, _H), lambda b: (b, 0, 0)),
            pl.BlockSpec((_NB, _S, _H), lambda b: (b, 0, 0)),
            pl.BlockSpec((_NB, 1, _DP), lambda b: (b, 0, 0)),
            pl.BlockSpec((_NB, 1, _DP), lambda b: (b, 0, 0)),
            pl.BlockSpec((_NB, 1, _DP), lambda b: (b, 0, 0)),
            pl.BlockSpec((_NB, 1, _DP), lambda b: (b, 0, 0)),
        ],
        out_shape=[
            jax.ShapeDtypeStruct((_B, _S, _H), jnp.float32),
            jax.ShapeDtypeStruct((_B, _S, _H), jnp.float32),
            ws_sds, ws_sds, ws_sds, ws_sds,
        ],
        compiler_params=pltpu.CompilerParams(
            dimension_semantics=("arbitrary",)),
        name="attn_pool",
    )(seg_features, w_all, b_all)

    probs, idc, closs = pl.pallas_call(
        _head_kernel,
        in_specs=[
            pl.BlockSpec((_B, 1, _DP), lambda: (0, 0, 0)),
            pl.BlockSpec((_B, 1, _DP), lambda: (0, 0, 0)),
            pl.BlockSpec((_B, 1, _DP), lambda: (0, 0, 0)),
            pl.BlockSpec((_B, 1, _DP), lambda: (0, 0, 0)),
            pl.BlockSpec((_DP, _C), lambda: (0, 0)),
            pl.BlockSpec((1, _C), lambda: (0, 0)),
            pl.BlockSpec((1, _C), lambda: (0, 0)),
            pl.BlockSpec((1, _C), lambda: (0, 0)),
        ],
        out_specs=[
            pl.BlockSpec((_B, _C), lambda: (0, 0)),
            pl.BlockSpec((_B, _C), lambda: (0, 0)),
            pl.BlockSpec((1, 1, 1), lambda: (0, 0, 0)),
        ],
        out_shape=[
            jax.ShapeDtypeStruct((_B, _C), jnp.float32),
            jax.ShapeDtypeStruct((_B, _C), jnp.int32),
            jax.ShapeDtypeStruct((1, 1, 1), jnp.float32),
        ],
        name="classifier_head",
    )(ws0, ws1, ws2, ws3, wct, bc2, lna2, lnb2)

    vid_probs = probs
    attn_idc = idc
    conv_loss = closs[0, 0, 0]
    return (vid_probs, attn_idc, scores_b, attnw_b, conv_loss)


# speedup vs baseline: 1.2520x; 1.0194x over previous
import math

import jax
import jax.numpy as jnp
from jax.experimental import pallas as pl
from jax.experimental.pallas import tpu as pltpu

_B, _S, _DM, _DP, _H, _C = 64, 300, 1024, 128, 4, 3862
_NB = 4


def _pool_kernel(seg_ref, w_ref, b_ref,
                 scores_ref, attnw_ref, ws0_ref, ws1_ref, ws2_ref, ws3_ref):
    scale = 1.0 / math.sqrt(_DP)
    ws_refs = (ws0_ref, ws1_ref, ws2_ref, ws3_ref)
    for i in range(_NB):
        seg = seg_ref[i].astype(jnp.bfloat16)
        proj = jnp.dot(seg, w_ref[...],
                       preferred_element_type=jnp.float32) + b_ref[...]
        k = proj[:, 4 * _DP:5 * _DP]
        v = proj[:, 5 * _DP:6 * _DP]

        kb = k.astype(jnp.bfloat16).astype(jnp.float32)
        for h in range(_H):
            qh = proj[:, h * _DP:(h + 1) * _DP]
            qb = qh.astype(jnp.bfloat16).astype(jnp.float32)
            sh = jnp.sum(qb * kb, axis=1, keepdims=True) * scale
            scores_ref[i, :, h:h + 1] = sh
        score = scores_ref[i]

        m = jnp.max(score, axis=0, keepdims=True)
        e = jnp.exp(score - m)
        p = e / jnp.sum(e, axis=0, keepdims=True)
        attnw_ref[i] = p

        ws = jax.lax.dot_general(p, v, (((0,), (0,)), ((), ())),
                                 preferred_element_type=jnp.float32)
        ws = jnp.maximum(ws, 0.0)
        for h in range(_H):
            ws_refs[h][i] = ws[h:h + 1, :]


def _head_kernel(ws0_ref, ws1_ref, ws2_ref, ws3_ref, wct_ref, bc_ref,
                 lna_ref, lnb_ref, probs_ref, idc_ref, closs_ref):
    wct = wct_ref[...]
    bc = bc_ref[...]
    lna = lna_ref[...]
    lnb = lnb_ref[...]
    vmax = None
    idc = None
    for h, ws_ref in enumerate((ws0_ref, ws1_ref, ws2_ref, ws3_ref)):
        wsr = ws_ref[...].reshape(-1, _DP)
        logits = jnp.dot(wsr, wct,
                         preferred_element_type=jnp.float32) + bc
        mean = jnp.mean(logits, axis=1, keepdims=True)
        xc = logits - mean
        var = jnp.sum(xc * xc, axis=1, keepdims=True) / (_C - 1)
        std = jnp.sqrt(var)
        ln = lna * xc / (std + 1e-6) + lnb
        if h == 0:
            vmax = ln
            idc = jnp.zeros(ln.shape, jnp.int32)
        else:
            gt = ln > vmax
            vmax = jnp.where(gt, ln, vmax)
            idc = jnp.where(gt, h, idc)
    probs_ref[...] = jax.nn.sigmoid(vmax)
    idc_ref[...] = idc

    wsum = jnp.sum(wct, axis=0, keepdims=True) + bc
    cm = jnp.max(wsum, axis=1, keepdims=True)
    ce = jnp.exp(wsum - cm)
    cp = ce / jnp.sum(ce, axis=1, keepdims=True)
    cmean = jnp.mean(cp, axis=1, keepdims=True)
    cd = cp - cmean
    cstd = jnp.sqrt(jnp.sum(cd * cd, axis=1, keepdims=True) / (_C - 1))
    closs_ref[...] = (float(_B) * jnp.clip(cstd, 1e-9, 1e9)).reshape(1, 1, 1)


def kernel(seg_features, Wq, bq, Wk, bk, Wv, bv, Wc, bc, ln_a, ln_b):
    w_all = jnp.concatenate([Wq.reshape(_H * _DP, _DM), Wk, Wv],
                            axis=0).T.astype(jnp.bfloat16)
    b_all = jnp.concatenate([bq.reshape(_H * _DP), bk, bv]).reshape(1, 6 * _DP)
    wct = Wc.T
    bc2 = bc.reshape(1, _C)
    lna2 = ln_a.reshape(1, _C)
    lnb2 = ln_b.reshape(1, _C)

    ws_sds = jax.ShapeDtypeStruct((_B, 1, _DP), jnp.float32)
    scores_b, attnw_b, ws0, ws1, ws2, ws3 = pl.pallas_call(
        _pool_kernel,
        grid=(_B // _NB,),
        in_specs=[
            pl.BlockSpec((_NB, _S, _DM), lambda b: (b, 0, 0)),
            pl.BlockSpec((_DM, 6 * _DP), lambda b: (0, 0)),
            pl.BlockSpec((1, 6 * _DP), lambda b: (0, 0)),
        ],
        out_specs=[
            pl.BlockSpec((_NB, _S, _H), lambda b: (b, 0, 0)),
            pl.BlockSpec((_NB, _S, _H), lambda b: (b, 0, 0)),
            pl.BlockSpec((_NB, 1, _DP), lambda b: (b, 0, 0)),
            pl.BlockSpec((_NB, 1, _DP), lambda b: (b, 0, 0)),
            pl.BlockSpec((_NB, 1, _DP), lambda b: (b, 0, 0)),
            pl.BlockSpec((_NB, 1, _DP), lambda b: (b, 0, 0)),
        ],
        out_shape=[
            jax.ShapeDtypeStruct((_B, _S, _H), jnp.float32),
            jax.ShapeDtypeStruct((_B, _S, _H), jnp.float32),
            ws_sds, ws_sds, ws_sds, ws_sds,
        ],
        compiler_params=pltpu.CompilerParams(
            dimension_semantics=("arbitrary",)),
        name="attn_pool",
    )(seg_features, w_all, b_all)

    probs, idc, closs = pl.pallas_call(
        _head_kernel,
        in_specs=[
            pl.BlockSpec((_B, 1, _DP), lambda: (0, 0, 0)),
            pl.BlockSpec((_B, 1, _DP), lambda: (0, 0, 0)),
            pl.BlockSpec((_B, 1, _DP), lambda: (0, 0, 0)),
            pl.BlockSpec((_B, 1, _DP), lambda: (0, 0, 0)),
            pl.BlockSpec((_DP, _C), lambda: (0, 0)),
            pl.BlockSpec((1, _C), lambda: (0, 0)),
            pl.BlockSpec((1, _C), lambda: (0, 0)),
            pl.BlockSpec((1, _C), lambda: (0, 0)),
        ],
        out_specs=[
            pl.BlockSpec((_B, _C), lambda: (0, 0)),
            pl.BlockSpec((_B, _C), lambda: (0, 0)),
            pl.BlockSpec((1, 1, 1), lambda: (0, 0, 0)),
        ],
        out_shape=[
            jax.ShapeDtypeStruct((_B, _C), jnp.float32),
            jax.ShapeDtypeStruct((_B, _C), jnp.int32),
            jax.ShapeDtypeStruct((1, 1, 1), jnp.float32),
        ],
        name="classifier_head",
    )(ws0, ws1, ws2, ws3, wct, bc2, lna2, lnb2)

    vid_probs = probs
    attn_idc = idc
    conv_loss = closs[0, 0, 0]
    return (vid_probs, attn_idc, scores_b, attnw_b, conv_loss)


# speedup vs baseline: 1.5671x; 1.2517x over previous
import math

import jax
import jax.numpy as jnp
from jax.experimental import pallas as pl
from jax.experimental.pallas import tpu as pltpu

_B, _S, _DM, _DP, _H, _C = 64, 300, 1024, 128, 4, 3862
_NB = 4


def _pool_kernel(seg_ref, w_ref, b_ref,
                 scores_ref, attnw_ref, ws0_ref, ws1_ref, ws2_ref, ws3_ref):
    scale = 1.0 / math.sqrt(_DP)
    ws_refs = (ws0_ref, ws1_ref, ws2_ref, ws3_ref)

    projs = []
    for i in range(_NB):
        seg = seg_ref[i].astype(jnp.bfloat16)
        projs.append(jnp.dot(seg, w_ref[...],
                             preferred_element_type=jnp.float32) + b_ref[...])

    scores = []
    for i in range(_NB):
        proj = projs[i]
        kb = (proj[:, 4 * _DP:5 * _DP]
              .astype(jnp.bfloat16).astype(jnp.float32))
        score = None
        for h in range(_H):
            qh = proj[:, h * _DP:(h + 1) * _DP]
            qb = qh.astype(jnp.bfloat16).astype(jnp.float32)
            sh = jnp.sum(qb * kb, axis=1, keepdims=True) * scale
            onehot = (jax.lax.broadcasted_iota(jnp.int32, (1, _H), 1) == h
                      ).astype(jnp.float32)
            contrib = sh * onehot
            score = contrib if score is None else score + contrib
        scores.append(score)
        scores_ref[i] = score

    ps = []
    for i in range(_NB):
        score = scores[i]
        m = jnp.max(score, axis=0, keepdims=True)
        e = jnp.exp(score - m)
        p = e / jnp.sum(e, axis=0, keepdims=True)
        ps.append(p)
        attnw_ref[i] = p

    for i in range(_NB):
        v = projs[i][:, 5 * _DP:6 * _DP]
        ws = jax.lax.dot_general(ps[i], v, (((0,), (0,)), ((), ())),
                                 preferred_element_type=jnp.float32)
        ws = jnp.maximum(ws, 0.0)
        for h in range(_H):
            ws_refs[h][i] = ws[h:h + 1, :]


def _head_kernel(ws0_ref, ws1_ref, ws2_ref, ws3_ref, wct_ref, bc_ref,
                 lna_ref, lnb_ref, probs_ref, idc_ref, closs_ref):
    wct = wct_ref[...]
    bc = bc_ref[...]
    lna = lna_ref[...]
    lnb = lnb_ref[...]
    vmax = None
    idc = None
    for h, ws_ref in enumerate((ws0_ref, ws1_ref, ws2_ref, ws3_ref)):
        wsr = ws_ref[...].reshape(-1, _DP)
        logits = jnp.dot(wsr, wct,
                         preferred_element_type=jnp.float32) + bc
        mean = jnp.mean(logits, axis=1, keepdims=True)
        xc = logits - mean
        var = jnp.sum(xc * xc, axis=1, keepdims=True) / (_C - 1)
        std = jnp.sqrt(var)
        ln = lna * xc / (std + 1e-6) + lnb
        if h == 0:
            vmax = ln
            idc = jnp.zeros(ln.shape, jnp.int32)
        else:
            gt = ln > vmax
            vmax = jnp.where(gt, ln, vmax)
            idc = jnp.where(gt, h, idc)
    probs_ref[...] = jax.nn.sigmoid(vmax)
    idc_ref[...] = idc

    wsum = jnp.sum(wct, axis=0, keepdims=True) + bc
    cm = jnp.max(wsum, axis=1, keepdims=True)
    ce = jnp.exp(wsum - cm)
    cp = ce / jnp.sum(ce, axis=1, keepdims=True)
    cmean = jnp.mean(cp, axis=1, keepdims=True)
    cd = cp - cmean
    cstd = jnp.sqrt(jnp.sum(cd * cd, axis=1, keepdims=True) / (_C - 1))
    closs_ref[...] = (float(_B) * jnp.clip(cstd, 1e-9, 1e9)).reshape(1, 1, 1)


def kernel(seg_features, Wq, bq, Wk, bk, Wv, bv, Wc, bc, ln_a, ln_b):
    w_all = jnp.concatenate([Wq.reshape(_H * _DP, _DM), Wk, Wv],
                            axis=0).T.astype(jnp.bfloat16)
    b_all = jnp.concatenate([bq.reshape(_H * _DP), bk, bv]).reshape(1, 6 * _DP)
    wct = Wc.T
    bc2 = bc.reshape(1, _C)
    lna2 = ln_a.reshape(1, _C)
    lnb2 = ln_b.reshape(1, _C)

    ws_sds = jax.ShapeDtypeStruct((_B, 1, _DP), jnp.float32)
    scores_b, attnw_b, ws0, ws1, ws2, ws3 = pl.pallas_call(
        _pool_kernel,
        grid=(_B // _NB,),
        in_specs=[
            pl.BlockSpec((_NB, _S, _DM), lambda b: (b, 0, 0)),
            pl.BlockSpec((_DM, 6 * _DP), lambda b: (0, 0)),
            pl.BlockSpec((1, 6 * _DP), lambda b: (0, 0)),
        ],
        out_specs=[
            pl.BlockSpec((_NB, _S, _H), lambda b: (b, 0, 0)),
            pl.BlockSpec((_NB, _S, _H), lambda b: (b, 0, 0)),
            pl.BlockSpec((_NB, 1, _DP), lambda b: (b, 0, 0)),
            pl.BlockSpec((_NB, 1, _DP), lambda b: (b, 0, 0)),
            pl.BlockSpec((_NB, 1, _DP), lambda b: (b, 0, 0)),
            pl.BlockSpec((_NB, 1, _DP), lambda b: (b, 0, 0)),
        ],
        out_shape=[
            jax.ShapeDtypeStruct((_B, _S, _H), jnp.float32),
            jax.ShapeDtypeStruct((_B, _S, _H), jnp.float32),
            ws_sds, ws_sds, ws_sds, ws_sds,
        ],
        compiler_params=pltpu.CompilerParams(
            dimension_semantics=("arbitrary",)),
        name="attn_pool",
    )(seg_features, w_all, b_all)

    probs, idc, closs = pl.pallas_call(
        _head_kernel,
        in_specs=[
            pl.BlockSpec((_B, 1, _DP), lambda: (0, 0, 0)),
            pl.BlockSpec((_B, 1, _DP), lambda: (0, 0, 0)),
            pl.BlockSpec((_B, 1, _DP), lambda: (0, 0, 0)),
            pl.BlockSpec((_B, 1, _DP), lambda: (0, 0, 0)),
            pl.BlockSpec((_DP, _C), lambda: (0, 0)),
            pl.BlockSpec((1, _C), lambda: (0, 0)),
            pl.BlockSpec((1, _C), lambda: (0, 0)),
            pl.BlockSpec((1, _C), lambda: (0, 0)),
        ],
        out_specs=[
            pl.BlockSpec((_B, _C), lambda: (0, 0)),
            pl.BlockSpec((_B, _C), lambda: (0, 0)),
            pl.BlockSpec((1, 1, 1), lambda: (0, 0, 0)),
        ],
        out_shape=[
            jax.ShapeDtypeStruct((_B, _C), jnp.float32),
            jax.ShapeDtypeStruct((_B, _C), jnp.int32),
            jax.ShapeDtypeStruct((1, 1, 1), jnp.float32),
        ],
        name="classifier_head",
    )(ws0, ws1, ws2, ws3, wct, bc2, lna2, lnb2)

    vid_probs = probs
    attn_idc = idc
    conv_loss = closs[0, 0, 0]
    return (vid_probs, attn_idc, scores_b, attnw_b, conv_loss)


# speedup vs baseline: 1.6027x; 1.0227x over previous
import math

import jax
import jax.numpy as jnp
from jax.experimental import pallas as pl
from jax.experimental.pallas import tpu as pltpu

_B, _S, _DM, _DP, _H, _C = 64, 300, 1024, 128, 4, 3862
_NB = 8


def _pool_kernel(seg_ref, w_ref, b_ref,
                 scores_ref, attnw_ref, ws0_ref, ws1_ref, ws2_ref, ws3_ref):
    scale = 1.0 / math.sqrt(_DP)
    ws_refs = (ws0_ref, ws1_ref, ws2_ref, ws3_ref)

    projs = []
    for i in range(_NB):
        seg = seg_ref[i].astype(jnp.bfloat16)
        projs.append(jnp.dot(seg, w_ref[...],
                             preferred_element_type=jnp.float32) + b_ref[...])

    scores = []
    for i in range(_NB):
        proj = projs[i]
        kb = (proj[:, 4 * _DP:5 * _DP]
              .astype(jnp.bfloat16).astype(jnp.float32))
        lane = jax.lax.broadcasted_iota(jnp.int32, (1, _H), 1)
        shs = []
        for h in range(_H):
            qh = proj[:, h * _DP:(h + 1) * _DP]
            qb = qh.astype(jnp.bfloat16).astype(jnp.float32)
            shs.append(jnp.sum(qb * kb, axis=1, keepdims=True) * scale)
        score = jnp.where(lane == 0, shs[0],
                          jnp.where(lane == 1, shs[1],
                                    jnp.where(lane == 2, shs[2], shs[3])))
        scores.append(score)
        scores_ref[i] = score

    ps = []
    for i in range(_NB):
        score = scores[i]
        m = jnp.max(score, axis=0, keepdims=True)
        e = jnp.exp(score - m)
        p = e / jnp.sum(e, axis=0, keepdims=True)
        ps.append(p)
        attnw_ref[i] = p

    for i in range(_NB):
        v = projs[i][:, 5 * _DP:6 * _DP]
        ws = jax.lax.dot_general(ps[i], v, (((0,), (0,)), ((), ())),
                                 preferred_element_type=jnp.float32)
        ws = jnp.maximum(ws, 0.0)
        for h in range(_H):
            ws_refs[h][i] = ws[h:h + 1, :]


def _head_kernel(ws0_ref, ws1_ref, ws2_ref, ws3_ref, wct_ref, bc_ref,
                 lna_ref, lnb_ref, probs_ref, idc_ref, closs_ref):
    wct = wct_ref[...]
    wctb = wct.astype(jnp.bfloat16)
    bc = bc_ref[...]
    lna = lna_ref[...]
    lnb = lnb_ref[...]
    vmax = None
    idc = None
    for h, ws_ref in enumerate((ws0_ref, ws1_ref, ws2_ref, ws3_ref)):
        wsr = ws_ref[...].reshape(-1, _DP).astype(jnp.bfloat16)
        logits = jnp.dot(wsr, wctb,
                         preferred_element_type=jnp.float32) + bc
        mean = jnp.mean(logits, axis=1, keepdims=True)
        xc = logits - mean
        var = jnp.sum(xc * xc, axis=1, keepdims=True) / (_C - 1)
        std = jnp.sqrt(var)
        ln = lna * xc / (std + 1e-6) + lnb
        if h == 0:
            vmax = ln
            idc = jnp.zeros(ln.shape, jnp.int32)
        else:
            gt = ln > vmax
            vmax = jnp.where(gt, ln, vmax)
            idc = jnp.where(gt, h, idc)
    probs_ref[...] = jax.nn.sigmoid(vmax)
    idc_ref[...] = idc

    wsum = jnp.sum(wct, axis=0, keepdims=True) + bc
    cm = jnp.max(wsum, axis=1, keepdims=True)
    ce = jnp.exp(wsum - cm)
    cp = ce / jnp.sum(ce, axis=1, keepdims=True)
    cmean = jnp.mean(cp, axis=1, keepdims=True)
    cd = cp - cmean
    cstd = jnp.sqrt(jnp.sum(cd * cd, axis=1, keepdims=True) / (_C - 1))
    closs_ref[...] = (float(_B) * jnp.clip(cstd, 1e-9, 1e9)).reshape(1, 1, 1)


def kernel(seg_features, Wq, bq, Wk, bk, Wv, bv, Wc, bc, ln_a, ln_b):
    w_all = jnp.concatenate([Wq.reshape(_H * _DP, _DM), Wk, Wv],
                            axis=0).T.astype(jnp.bfloat16)
    b_all = jnp.concatenate([bq.reshape(_H * _DP), bk, bv]).reshape(1, 6 * _DP)
    wct = Wc.T
    bc2 = bc.reshape(1, _C)
    lna2 = ln_a.reshape(1, _C)
    lnb2 = ln_b.reshape(1, _C)

    ws_sds = jax.ShapeDtypeStruct((_B, 1, _DP), jnp.float32)
    scores_b, attnw_b, ws0, ws1, ws2, ws3 = pl.pallas_call(
        _pool_kernel,
        grid=(_B // _NB,),
        in_specs=[
            pl.BlockSpec((_NB, _S, _DM), lambda b: (b, 0, 0)),
            pl.BlockSpec((_DM, 6 * _DP), lambda b: (0, 0)),
            pl.BlockSpec((1, 6 * _DP), lambda b: (0, 0)),
        ],
        out_specs=[
            pl.BlockSpec((_NB, _S, _H), lambda b: (b, 0, 0)),
            pl.BlockSpec((_NB, _S, _H), lambda b: (b, 0, 0)),
            pl.BlockSpec((_NB, 1, _DP), lambda b: (b, 0, 0)),
            pl.BlockSpec((_NB, 1, _DP), lambda b: (b, 0, 0)),
            pl.BlockSpec((_NB, 1, _DP), lambda b: (b, 0, 0)),
            pl.BlockSpec((_NB, 1, _DP), lambda b: (b, 0, 0)),
        ],
        out_shape=[
            jax.ShapeDtypeStruct((_B, _S, _H), jnp.float32),
            jax.ShapeDtypeStruct((_B, _S, _H), jnp.float32),
            ws_sds, ws_sds, ws_sds, ws_sds,
        ],
        compiler_params=pltpu.CompilerParams(
            dimension_semantics=("arbitrary",)),
        name="attn_pool",
    )(seg_features, w_all, b_all)

    probs, idc, closs = pl.pallas_call(
        _head_kernel,
        in_specs=[
            pl.BlockSpec((_B, 1, _DP), lambda: (0, 0, 0)),
            pl.BlockSpec((_B, 1, _DP), lambda: (0, 0, 0)),
            pl.BlockSpec((_B, 1, _DP), lambda: (0, 0, 0)),
            pl.BlockSpec((_B, 1, _DP), lambda: (0, 0, 0)),
            pl.BlockSpec((_DP, _C), lambda: (0, 0)),
            pl.BlockSpec((1, _C), lambda: (0, 0)),
            pl.BlockSpec((1, _C), lambda: (0, 0)),
            pl.BlockSpec((1, _C), lambda: (0, 0)),
        ],
        out_specs=[
            pl.BlockSpec((_B, _C), lambda: (0, 0)),
            pl.BlockSpec((_B, _C), lambda: (0, 0)),
            pl.BlockSpec((1, 1, 1), lambda: (0, 0, 0)),
        ],
        out_shape=[
            jax.ShapeDtypeStruct((_B, _C), jnp.float32),
            jax.ShapeDtypeStruct((_B, _C), jnp.int32),
            jax.ShapeDtypeStruct((1, 1, 1), jnp.float32),
        ],
        name="classifier_head",
    )(ws0, ws1, ws2, ws3, wct, bc2, lna2, lnb2)

    vid_probs = probs
    attn_idc = idc
    conv_loss = closs[0, 0, 0]
    return (vid_probs, attn_idc, scores_b, attnw_b, conv_loss)
